# baseline (speedup 1.0000x reference)
import jax
import jax.numpy as jnp
from jax.experimental import pallas as pl

_BLK = 32768


def _neg_body(x_ref, o_ref):
    o_ref[...] = -x_ref[...]


def kernel(x):
    b, d = x.shape
    grid = (d // _BLK,)
    re = pl.pallas_call(
        _neg_body,
        grid=grid,
        in_specs=[pl.BlockSpec((b, _BLK), lambda j: (0, j))],
        out_specs=pl.BlockSpec((b, _BLK), lambda j: (0, j)),
        out_shape=jax.ShapeDtypeStruct((b, d), jnp.float32),
    )(x)
    sq = re.reshape(4096, 4096)
    return jax.lax.complex(sq, jnp.zeros_like(sq))


# speedup vs baseline: 1.2649x; 1.2649x over previous
import numpy as np
import jax
import jax.numpy as jnp
from jax.sharding import Mesh, NamedSharding, PartitionSpec as P
from jax.experimental import pallas as pl

_BLK = 32768


def _neg_body(x_ref, o_ref):
    o_ref[...] = -x_ref[...]


def _local(x):
    b, d = x.shape
    grid = (d // _BLK,)
    re = pl.pallas_call(
        _neg_body,
        grid=grid,
        in_specs=[pl.BlockSpec((b, _BLK), lambda j: (0, j))],
        out_specs=pl.BlockSpec((b, _BLK), lambda j: (0, j)),
        out_shape=jax.ShapeDtypeStruct((b, d), jnp.float32),
    )(x)
    return jax.lax.complex(re, jnp.zeros_like(re))


def kernel(x):
    devs = jax.devices()
    n = 2 if len(devs) >= 2 else 1
    if n == 1 or x.shape[1] % (n * _BLK):
        return _local(x)
    mesh = Mesh(np.array(devs[:n]), ("d",))
    xs = jax.lax.with_sharding_constraint(x, NamedSharding(mesh, P(None, "d")))
    f = jax.shard_map(_local, mesh=mesh, in_specs=P(None, "d"),
                      out_specs=P(None, "d"), check_vma=False)
    return f(xs)


# speedup vs baseline: 1.9165x; 1.5151x over previous
import numpy as np
import jax
import jax.numpy as jnp
from jax.sharding import Mesh, NamedSharding, PartitionSpec as P
from jax.experimental import pallas as pl

_BLK = 32768


def _neg_body(x_ref, o_ref):
    o_ref[...] = -x_ref[...]


def _local(x):
    b, d = x.shape
    grid = (d // _BLK,)
    re = pl.pallas_call(
        _neg_body,
        grid=grid,
        in_specs=[pl.BlockSpec((b, _BLK), lambda j: (0, j))],
        out_specs=pl.BlockSpec((b, _BLK), lambda j: (0, j)),
        out_shape=jax.ShapeDtypeStruct((b, d), jnp.float32),
    )(x)
    return jax.lax.complex(re, jnp.zeros_like(re))


def _local_sliced(x):
    b, d = x.shape
    half = d // 2
    i = jax.lax.axis_index("d")
    xloc = jax.lax.dynamic_slice(x, (0, i * half), (b, half))
    return _local(xloc)


def kernel(x):
    devs = jax.devices()
    n = 2 if len(devs) >= 2 else 1
    if n == 1 or x.shape[1] % (n * _BLK):
        return _local(x)
    mesh = Mesh(np.array(devs[:n]), ("d",))
    xr = jax.lax.with_sharding_constraint(x, NamedSharding(mesh, P()))
    f = jax.shard_map(_local_sliced, mesh=mesh, in_specs=P(),
                      out_specs=P(None, "d"), check_vma=False)
    return f(xr)


# speedup vs baseline: 2.0633x; 1.0766x over previous
import numpy as np
import jax
import jax.numpy as jnp
from jax.sharding import Mesh, NamedSharding, PartitionSpec as P
from jax.experimental import pallas as pl
from jax.experimental.pallas import tpu as pltpu

_BLK = 32768


def _neg_body(off_ref, x_ref, o_ref):
    del off_ref
    o_ref[...] = -x_ref[...]


def _neg_lanes(x, off_blocks, out_lanes):
    b = x.shape[0]
    grid = (out_lanes // _BLK,)
    return pl.pallas_call(
        _neg_body,
        grid_spec=pltpu.PrefetchScalarGridSpec(
            num_scalar_prefetch=1,
            grid=grid,
            in_specs=[pl.BlockSpec((b, _BLK), lambda j, off: (0, off[0] + j))],
            out_specs=pl.BlockSpec((b, _BLK), lambda j, off: (0, j)),
        ),
        out_shape=jax.ShapeDtypeStruct((b, out_lanes), jnp.float32),
    )(off_blocks, x)


def _complexify(re):
    return jax.lax.complex(re, jnp.zeros_like(re))


def _local_full(x):
    off = jnp.zeros((1,), jnp.int32)
    return _complexify(_neg_lanes(x, off, x.shape[1]))


def _make_sharded(n):
    def _local_shard(x):
        d = x.shape[1]
        part = d // n
        i = jax.lax.axis_index("d")
        off = (i * (part // _BLK)).astype(jnp.int32).reshape(1)
        return _complexify(_neg_lanes(x, off, part))

    return _local_shard


def kernel(x):
    devs = jax.devices()
    n = 1
    while (n * 2 <= len(devs)) and (x.shape[1] % (n * 2 * _BLK) == 0):
        n *= 2
    if n == 1:
        return _local_full(x)
    mesh = Mesh(np.array(devs[:n]), ("d",))
    xr = jax.lax.with_sharding_constraint(x, NamedSharding(mesh, P()))
    f = jax.shard_map(_make_sharded(n), mesh=mesh, in_specs=P(),
                      out_specs=P(None, "d"), check_vma=False)
    return f(xr)
